# Initial kernel scaffold; baseline (speedup 1.0000x reference)
import jax
import jax.numpy as jnp
from jax.experimental import pallas as pl


def kernel(input, embed):
    raise NotImplementedError("write your pallas kernel here")


# speedup vs baseline: 1.3461x; 1.3461x over previous
import functools

import jax
import jax.numpy as jnp
from jax import lax
from jax.experimental import pallas as pl
from jax.experimental.pallas import tpu as pltpu
from jax.experimental.pallas import tpu_sc as plsc

_DIM = 256
_NE = 8192
_BM = 256
_CH = 2048


def _dist_body(fsq_ref, ft_ref, et_ref, esq_ref, idx_ref, dsum_ref):
    i = pl.program_id(0)
    s = jnp.dot(et_ref[...], ft_ref[...], preferred_element_type=jnp.float32)
    dist = fsq_ref[...] - 2.0 * s + esq_ref[...]

    v = None
    idx = None
    for c in range(0, _NE, _CH):
        blk = dist[c:c + _CH]
        li = jnp.argmin(blk, axis=0).astype(jnp.int32) + c
        lv = jnp.min(blk, axis=0)
        if v is None:
            v, idx = lv, li
        else:
            take_old = (v < lv) | ((v == lv) & (idx < li))
            v = jnp.where(take_old, v, lv)
            idx = jnp.where(take_old, idx, li)
        v = v.astype(jnp.bfloat16).astype(jnp.float32)

    idx_ref[...] = idx.reshape(1, _BM)

    @pl.when(i == 0)
    def _init():
        dsum_ref[...] = jnp.zeros((1, 1), jnp.float32)

    dsum_ref[...] += jnp.sum(jnp.min(dist, axis=0)).reshape(1, 1)


def _argmin_dist(ft, et, fsq_row, esq_col):
    n = ft.shape[1]
    return pl.pallas_call(
        _dist_body,
        grid=(n // _BM,),
        in_specs=[
            pl.BlockSpec((1, _BM), lambda i: (0, i)),
            pl.BlockSpec((_DIM, _BM), lambda i: (0, i)),
            pl.BlockSpec((_NE, _DIM), lambda i: (0, 0)),
            pl.BlockSpec((_NE, 1), lambda i: (0, 0)),
        ],
        out_specs=[
            pl.BlockSpec((1, _BM), lambda i: (0, i)),
            pl.BlockSpec((1, 1), lambda i: (0, 0)),
        ],
        out_shape=[
            jax.ShapeDtypeStruct((1, n), jnp.int32),
            jax.ShapeDtypeStruct((1, 1), jnp.float32),
        ],
    )(fsq_row, ft, et, esq_col)


def _make_sc_gather(n_rows):
    info = plsc.get_sparse_core_info()
    nc, ns = info.num_cores, info.num_subcores
    nw = nc * ns
    bpw = n_rows // nw
    ch = 128
    nch = bpw // ch
    mesh = plsc.VectorSubcoreMesh(core_axis_name="c", subcore_axis_name="s")

    @functools.partial(
        pl.kernel,
        mesh=mesh,
        out_type=jax.ShapeDtypeStruct((n_rows, _DIM), jnp.float32),
        scratch_types=[
            pltpu.VMEM((ch,), jnp.int32),
            pltpu.VMEM((ch, _DIM), jnp.float32),
            pltpu.SemaphoreType.DMA,
        ],
    )
    def gather(table_hbm, idx_hbm, out_hbm, idx_v, rows_v, sem):
        wid = lax.axis_index("s") * nc + lax.axis_index("c")
        for c in range(nch):
            base = wid * bpw + c * ch
            pltpu.sync_copy(idx_hbm.at[pl.ds(base, ch)], idx_v)
            pltpu.async_copy(table_hbm.at[idx_v], rows_v, sem).wait()
            pltpu.sync_copy(rows_v, out_hbm.at[pl.ds(base, ch)])

    return gather


def kernel(input, embed):
    flatten = input.reshape(-1, _DIM)
    n = flatten.shape[0]
    fsq = jnp.sum(flatten**2, axis=1, keepdims=True)
    esq = jnp.sum(embed**2, axis=0, keepdims=True)
    ft = flatten.T.astype(jnp.bfloat16)
    et = embed.T
    idx2, dsum = _argmin_dist(ft, et, fsq.reshape(1, n), esq.reshape(_NE, 1))
    embed_ind = idx2.reshape(input.shape[:-1])
    quantize = _make_sc_gather(n)(et, idx2.reshape(-1))
    quantize = quantize.reshape(input.shape)
    diff = dsum[0, 0] / (n * _DIM)
    return (quantize, diff, embed_ind)


# speedup vs baseline: 1.4425x; 1.0716x over previous
import functools

import jax
import jax.numpy as jnp
from jax import lax
from jax.experimental import pallas as pl
from jax.experimental.pallas import tpu as pltpu
from jax.experimental.pallas import tpu_sc as plsc

_DIM = 256
_NE = 8192
_BM = 512
_CH = 2048


def _dist_body(fsq_ref, ft_ref, et_ref, esq_ref, idx_ref, dsum_ref):
    i = pl.program_id(0)
    s = jnp.dot(et_ref[...], ft_ref[...], preferred_element_type=jnp.float32)
    dist = fsq_ref[...] - 2.0 * s + esq_ref[...]

    v = None
    idx = None
    for c in range(0, _NE, _CH):
        blk = dist[c:c + _CH]
        li = jnp.argmin(blk, axis=0).astype(jnp.int32) + c
        lv = jnp.min(blk, axis=0)
        if v is None:
            v, idx = lv, li
        else:
            take_old = (v < lv) | ((v == lv) & (idx < li))
            v = jnp.where(take_old, v, lv)
            idx = jnp.where(take_old, idx, li)
        v = v.astype(jnp.bfloat16).astype(jnp.float32)

    idx_ref[...] = idx.reshape(1, _BM)

    @pl.when(i == 0)
    def _init():
        dsum_ref[...] = jnp.zeros((1, 1), jnp.float32)

    dsum_ref[...] += jnp.sum(jnp.min(dist, axis=0)).reshape(1, 1)


def _argmin_dist(ft, et, fsq_row, esq_col):
    n = ft.shape[1]
    return pl.pallas_call(
        _dist_body,
        grid=(n // _BM,),
        in_specs=[
            pl.BlockSpec((1, _BM), lambda i: (0, i)),
            pl.BlockSpec((_DIM, _BM), lambda i: (0, i)),
            pl.BlockSpec((_NE, _DIM), lambda i: (0, 0)),
            pl.BlockSpec((_NE, 1), lambda i: (0, 0)),
        ],
        out_specs=[
            pl.BlockSpec((1, _BM), lambda i: (0, i)),
            pl.BlockSpec((1, 1), lambda i: (0, 0)),
        ],
        out_shape=[
            jax.ShapeDtypeStruct((1, n), jnp.int32),
            jax.ShapeDtypeStruct((1, 1), jnp.float32),
        ],
    )(fsq_row, ft, et, esq_col)


def _make_sc_gather(n_rows):
    info = plsc.get_sparse_core_info()
    nc, ns = info.num_cores, info.num_subcores
    nw = nc * ns
    bpw = n_rows // nw
    ch = 128
    nch = bpw // ch
    mesh = plsc.VectorSubcoreMesh(core_axis_name="c", subcore_axis_name="s")

    @functools.partial(
        pl.kernel,
        mesh=mesh,
        out_type=jax.ShapeDtypeStruct((n_rows, _DIM), jnp.float32),
        scratch_types=[
            pltpu.VMEM((ch,), jnp.int32),
            pltpu.VMEM((ch, _DIM), jnp.float32),
            pltpu.SemaphoreType.DMA,
        ],
    )
    def gather(table_hbm, idx_hbm, out_hbm, idx_v, rows_v, sem):
        wid = lax.axis_index("s") * nc + lax.axis_index("c")
        for c in range(nch):
            base = wid * bpw + c * ch
            pltpu.sync_copy(idx_hbm.at[pl.ds(base, ch)], idx_v)
            pltpu.async_copy(table_hbm.at[idx_v], rows_v, sem).wait()
            pltpu.sync_copy(rows_v, out_hbm.at[pl.ds(base, ch)])

    return gather


def kernel(input, embed):
    flatten = input.reshape(-1, _DIM)
    n = flatten.shape[0]
    fsq = jnp.sum(flatten**2, axis=1, keepdims=True)
    esq = jnp.sum(embed**2, axis=0, keepdims=True)
    ft = flatten.T.astype(jnp.bfloat16)
    et = embed.T
    idx2, dsum = _argmin_dist(ft, et, fsq.reshape(1, n), esq.reshape(_NE, 1))
    embed_ind = idx2.reshape(input.shape[:-1])
    quantize = _make_sc_gather(n)(et, idx2.reshape(-1))
    quantize = quantize.reshape(input.shape)
    diff = dsum[0, 0] / (n * _DIM)
    return (quantize, diff, embed_ind)


# speedup vs baseline: 1.5439x; 1.0703x over previous
import functools

import jax
import jax.numpy as jnp
from jax import lax
from jax.experimental import pallas as pl
from jax.experimental.pallas import tpu as pltpu
from jax.experimental.pallas import tpu_sc as plsc

_DIM = 256
_NE = 8192
_BM = 1024
_CH = 2048


def _dist_body(fsq_ref, ft_ref, et_ref, esq_ref, idx_ref, dsum_ref):
    i = pl.program_id(0)
    s = jnp.dot(et_ref[...], ft_ref[...], preferred_element_type=jnp.float32)
    dist = fsq_ref[...] - 2.0 * s + esq_ref[...]

    v = None
    idx = None
    for c in range(0, _NE, _CH):
        blk = dist[c:c + _CH]
        li = jnp.argmin(blk, axis=0).astype(jnp.int32) + c
        lv = jnp.min(blk, axis=0)
        if v is None:
            v, idx = lv, li
        else:
            take_old = (v < lv) | ((v == lv) & (idx < li))
            v = jnp.where(take_old, v, lv)
            idx = jnp.where(take_old, idx, li)
        v = v.astype(jnp.bfloat16).astype(jnp.float32)

    idx_ref[...] = idx.reshape(1, _BM)

    @pl.when(i == 0)
    def _init():
        dsum_ref[...] = jnp.zeros((1, 1), jnp.float32)

    dsum_ref[...] += jnp.sum(jnp.min(dist, axis=0)).reshape(1, 1)


def _argmin_dist(ft, et, fsq_row, esq_col):
    n = ft.shape[1]
    return pl.pallas_call(
        _dist_body,
        grid=(n // _BM,),
        in_specs=[
            pl.BlockSpec((1, _BM), lambda i: (0, i)),
            pl.BlockSpec((_DIM, _BM), lambda i: (0, i)),
            pl.BlockSpec((_NE, _DIM), lambda i: (0, 0)),
            pl.BlockSpec((_NE, 1), lambda i: (0, 0)),
        ],
        out_specs=[
            pl.BlockSpec((1, _BM), lambda i: (0, i)),
            pl.BlockSpec((1, 1), lambda i: (0, 0)),
        ],
        out_shape=[
            jax.ShapeDtypeStruct((1, n), jnp.int32),
            jax.ShapeDtypeStruct((1, 1), jnp.float32),
        ],
    )(fsq_row, ft, et, esq_col)


def _make_sc_gather(n_rows):
    info = plsc.get_sparse_core_info()
    nc, ns = info.num_cores, info.num_subcores
    nw = nc * ns
    bpw = n_rows // nw
    ch = 128
    nch = bpw // ch
    mesh = plsc.VectorSubcoreMesh(core_axis_name="c", subcore_axis_name="s")

    @functools.partial(
        pl.kernel,
        mesh=mesh,
        out_type=jax.ShapeDtypeStruct((n_rows, _DIM), jnp.float32),
        scratch_types=[
            pltpu.VMEM((ch,), jnp.int32),
            pltpu.VMEM((ch, _DIM), jnp.float32),
            pltpu.SemaphoreType.DMA,
        ],
    )
    def gather(table_hbm, idx_hbm, out_hbm, idx_v, rows_v, sem):
        wid = lax.axis_index("s") * nc + lax.axis_index("c")
        for c in range(nch):
            base = wid * bpw + c * ch
            pltpu.sync_copy(idx_hbm.at[pl.ds(base, ch)], idx_v)
            pltpu.async_copy(table_hbm.at[idx_v], rows_v, sem).wait()
            pltpu.sync_copy(rows_v, out_hbm.at[pl.ds(base, ch)])

    return gather


def kernel(input, embed):
    flatten = input.reshape(-1, _DIM)
    n = flatten.shape[0]
    fsq = jnp.sum(flatten**2, axis=1, keepdims=True)
    esq = jnp.sum(embed**2, axis=0, keepdims=True)
    ft = flatten.T.astype(jnp.bfloat16)
    et = embed.T
    idx2, dsum = _argmin_dist(ft, et, fsq.reshape(1, n), esq.reshape(_NE, 1))
    embed_ind = idx2.reshape(input.shape[:-1])
    quantize = _make_sc_gather(n)(et, idx2.reshape(-1))
    quantize = quantize.reshape(input.shape)
    diff = dsum[0, 0] / (n * _DIM)
    return (quantize, diff, embed_ind)
